# speedup vs baseline: 1.4803x; 1.4803x over previous
import functools

import jax
import jax.numpy as jnp
from jax import lax
from jax.experimental import pallas as pl
from jax.experimental.pallas import tpu as pltpu
from jax.experimental.pallas import tpu_sc as plsc

HIDDEN = 2048
NUM_CORES = 2
NUM_SUBCORES = 16
NUM_WORKERS = NUM_CORES * NUM_SUBCORES
CHUNK = 16


@functools.partial(jax.jit, static_argnums=(2,))
def _lookup(ids_flat, table, num_ids):
  b_per_w = num_ids // NUM_WORKERS
  n_chunks = b_per_w // CHUNK
  mesh = plsc.VectorSubcoreMesh(
      core_axis_name="c", subcore_axis_name="s", num_cores=NUM_CORES)

  @functools.partial(
      pl.kernel,
      mesh=mesh,
      out_type=jax.ShapeDtypeStruct((num_ids, HIDDEN), jnp.float32),
      scratch_types=[
          pltpu.VMEM((b_per_w,), jnp.int32),
          pltpu.VMEM((CHUNK, HIDDEN), jnp.float32),
          pltpu.SemaphoreType.DMA,
      ],
  )
  def k(idx_hbm, table_hbm, out_hbm, idx_v, rows_v, gsem):
    wid = lax.axis_index("s") * NUM_CORES + lax.axis_index("c")
    base = wid * b_per_w
    pltpu.sync_copy(idx_hbm.at[pl.ds(base, b_per_w)], idx_v)

    def body(g, carry):
      pltpu.async_copy(
          table_hbm.at[idx_v.at[pl.ds(g * CHUNK, CHUNK)]], rows_v, gsem
      ).wait()
      pltpu.sync_copy(rows_v, out_hbm.at[pl.ds(base + g * CHUNK, CHUNK)])
      return carry

    lax.fori_loop(0, n_chunks, body, 0)

  return k(ids_flat, table)


def kernel(input_ids, vocab_embedding):
  b, s = input_ids.shape
  ids_flat = input_ids.reshape(b * s).astype(jnp.int32)
  out = _lookup(ids_flat, vocab_embedding, b * s)
  return out.reshape(b, s, HIDDEN)


# speedup vs baseline: 1.7725x; 1.1974x over previous
import functools

import jax
import jax.numpy as jnp
from jax import lax
from jax.experimental import pallas as pl
from jax.experimental.pallas import tpu as pltpu
from jax.experimental.pallas import tpu_sc as plsc

HIDDEN = 2048
NUM_CORES = 2
NUM_SUBCORES = 16
NUM_WORKERS = NUM_CORES * NUM_SUBCORES
CHUNK = 8
NBUF = 4


@functools.partial(jax.jit, static_argnums=(2,))
def _lookup(ids_flat, table, num_ids):
  b_per_w = num_ids // NUM_WORKERS
  n_chunks = b_per_w // CHUNK
  n_rounds = n_chunks // NBUF
  mesh = plsc.VectorSubcoreMesh(
      core_axis_name="c", subcore_axis_name="s", num_cores=NUM_CORES)

  @functools.partial(
      pl.kernel,
      mesh=mesh,
      out_type=jax.ShapeDtypeStruct((num_ids, HIDDEN), jnp.float32),
      scratch_types=[
          pltpu.VMEM((b_per_w,), jnp.int32),
          pltpu.VMEM((NBUF, CHUNK, HIDDEN), jnp.float32),
          [pltpu.SemaphoreType.DMA] * NBUF,
          [pltpu.SemaphoreType.DMA] * NBUF,
      ],
  )
  def k(idx_hbm, table_hbm, out_hbm, idx_v, rows_v, gsems, ssems):
    wid = lax.axis_index("s") * NUM_CORES + lax.axis_index("c")
    base = wid * b_per_w
    pltpu.sync_copy(idx_hbm.at[pl.ds(base, b_per_w)], idx_v)

    def gather_start(g, slot):
      pltpu.async_copy(
          table_hbm.at[idx_v.at[pl.ds(g * CHUNK, CHUNK)]],
          rows_v.at[slot], gsems[slot])

    def gather_wait(slot):
      pltpu.make_async_copy(
          table_hbm.at[idx_v.at[pl.ds(0, CHUNK)]],
          rows_v.at[slot], gsems[slot]).wait()

    def store_start(g, slot):
      pltpu.async_copy(
          rows_v.at[slot], out_hbm.at[pl.ds(base + g * CHUNK, CHUNK)],
          ssems[slot])

    def store_wait(slot):
      pltpu.make_async_copy(
          rows_v.at[slot], out_hbm.at[pl.ds(base, CHUNK)],
          ssems[slot]).wait()

    for b in range(NBUF - 1):
      gather_start(b, b)

    def round_body(r, carry):
      for b in range(NBUF):
        g = r * NBUF + b
        prev = (b - 1) % NBUF
        @pl.when(g > 0)
        def _():
          store_wait(prev)

        @pl.when(g + NBUF - 1 < n_chunks)
        def _():
          gather_start(g + NBUF - 1, prev)

        gather_wait(b)
        store_start(g, b)
      return carry

    lax.fori_loop(0, n_rounds, round_body, 0)
    store_wait((n_chunks - 1) % NBUF)

  return k(ids_flat, table)


def kernel(input_ids, vocab_embedding):
  b, s = input_ids.shape
  ids_flat = input_ids.reshape(b * s).astype(jnp.int32)
  out = _lookup(ids_flat, vocab_embedding, b * s)
  return out.reshape(b, s, HIDDEN)


# speedup vs baseline: 1.7764x; 1.0022x over previous
import functools

import jax
import jax.numpy as jnp
from jax import lax
from jax.experimental import pallas as pl
from jax.experimental.pallas import tpu as pltpu
from jax.experimental.pallas import tpu_sc as plsc

HIDDEN = 2048
NUM_CORES = 2
NUM_SUBCORES = 16
NUM_WORKERS = NUM_CORES * NUM_SUBCORES
CHUNK = 8
NBUF = 6
LAG = 2
DEPTH = NBUF - LAG


@functools.partial(jax.jit, static_argnums=(2,))
def _lookup(ids_flat, table, num_ids):
  b_per_w = num_ids // NUM_WORKERS
  n_chunks = b_per_w // CHUNK
  n_rounds = n_chunks // NBUF
  tail = n_chunks - n_rounds * NBUF
  mesh = plsc.VectorSubcoreMesh(
      core_axis_name="c", subcore_axis_name="s", num_cores=NUM_CORES)

  @functools.partial(
      pl.kernel,
      mesh=mesh,
      out_type=jax.ShapeDtypeStruct((num_ids, HIDDEN), jnp.float32),
      scratch_types=[
          pltpu.VMEM((b_per_w,), jnp.int32),
          pltpu.VMEM((NBUF, CHUNK, HIDDEN), jnp.float32),
          [pltpu.SemaphoreType.DMA] * NBUF,
          [pltpu.SemaphoreType.DMA] * NBUF,
      ],
  )
  def k(idx_hbm, table_hbm, out_hbm, idx_v, rows_v, gsems, ssems):
    wid = lax.axis_index("s") * NUM_CORES + lax.axis_index("c")
    base = wid * b_per_w
    pltpu.sync_copy(idx_hbm.at[pl.ds(base, b_per_w)], idx_v)

    def gather_start(g, slot):
      pltpu.async_copy(
          table_hbm.at[idx_v.at[pl.ds(g * CHUNK, CHUNK)]],
          rows_v.at[slot], gsems[slot])

    def gather_wait(slot):
      pltpu.make_async_copy(
          table_hbm.at[idx_v.at[pl.ds(0, CHUNK)]],
          rows_v.at[slot], gsems[slot]).wait()

    def store_start(g, slot):
      pltpu.async_copy(
          rows_v.at[slot], out_hbm.at[pl.ds(base + g * CHUNK, CHUNK)],
          ssems[slot])

    def store_wait(slot):
      pltpu.make_async_copy(
          rows_v.at[slot], out_hbm.at[pl.ds(base, CHUNK)],
          ssems[slot]).wait()

    def emit_chunk(g, b, do_store_wait, do_gather_start):
      if do_store_wait:
        store_wait((b - LAG) % NBUF)
      if do_gather_start:
        gather_start(g + DEPTH, (b + DEPTH) % NBUF)
      gather_wait(b)
      store_start(g, b)

    for g in range(DEPTH):
      gather_start(g, g)

    for b in range(NBUF):
      emit_chunk(b, b, b >= LAG, b + DEPTH < n_chunks)

    def round_body(r, carry):
      g0 = r * NBUF
      for b in range(NBUF):
        emit_chunk(g0 + b, b, True, True)
      return carry

    lax.fori_loop(1, n_rounds, round_body, 0)

    for t in range(tail):
      g = n_rounds * NBUF + t
      emit_chunk(g, g % NBUF, True, g + DEPTH < n_chunks)

    for g in range(n_chunks - LAG, n_chunks):
      store_wait(g % NBUF)

  return k(ids_flat, table)


def kernel(input_ids, vocab_embedding):
  b, s = input_ids.shape
  ids_flat = input_ids.reshape(b * s).astype(jnp.int32)
  out = _lookup(ids_flat, vocab_embedding, b * s)
  return out.reshape(b, s, HIDDEN)
